# baseline (speedup 1.0000x reference)
import functools

import jax
import jax.numpy as jnp
from jax import lax
from jax.experimental import pallas as pl
from jax.experimental.pallas import tpu as pltpu
from jax.experimental.pallas import tpu_sc as plsc

N_NODES = 100000
EMB_DIM = 64

NUM_CORES = 2
NUM_SUBCORES = 16
NW = NUM_CORES * NUM_SUBCORES

CHUNK = 128
CHUNKS_PER_W = 25
ROWS_PER_W = CHUNK * CHUNKS_PER_W
B_PAD = NW * ROWS_PER_W

_mesh = plsc.VectorSubcoreMesh(core_axis_name="c", subcore_axis_name="s")


@functools.partial(
    pl.kernel,
    mesh=_mesh,
    compiler_params=pltpu.CompilerParams(use_tc_tiling_on_sc=False),
    out_type=jax.ShapeDtypeStruct((B_PAD, EMB_DIM), jnp.float32),
    scratch_types=[
        pltpu.VMEM((ROWS_PER_W,), jnp.int32),
        pltpu.VMEM((CHUNK, EMB_DIM), jnp.float32),
        pltpu.SemaphoreType.DMA,
    ],
)
def _gather_kernel(idx_hbm, table_hbm, out_hbm, idx_v, rows_v, gsem):
    wid = lax.axis_index("s") * NUM_CORES + lax.axis_index("c")
    base = wid * ROWS_PER_W
    pltpu.sync_copy(idx_hbm.at[pl.ds(base, ROWS_PER_W)], idx_v)

    def body(c, carry):
        pltpu.async_copy(
            table_hbm.at[idx_v.at[pl.ds(c * CHUNK, CHUNK)]], rows_v, gsem
        ).wait()
        pltpu.sync_copy(rows_v, out_hbm.at[pl.ds(base + c * CHUNK, CHUNK), :])
        return carry

    lax.fori_loop(0, CHUNKS_PER_W, body, 0)


def kernel(x, table):
    idx = x[:, 0]
    idx_pad = jnp.zeros((B_PAD,), jnp.int32).at[:N_NODES].set(idx)
    out = _gather_kernel(idx_pad, table)
    return out[:N_NODES]


# speedup vs baseline: 1.7063x; 1.7063x over previous
import functools

import jax
import jax.numpy as jnp
from jax import lax
from jax.experimental import pallas as pl
from jax.experimental.pallas import tpu as pltpu
from jax.experimental.pallas import tpu_sc as plsc

N_NODES = 100000
EMB_DIM = 64

NUM_CORES = 2
NUM_SUBCORES = 16
NW = NUM_CORES * NUM_SUBCORES

CHUNK = 128
ROWS_PER_W = 3128
NCH = 25
LAST_OFF = ROWS_PER_W - CHUNK
NBUF = 4

_mesh = plsc.VectorSubcoreMesh(core_axis_name="c", subcore_axis_name="s")


@functools.partial(
    pl.kernel,
    mesh=_mesh,
    compiler_params=pltpu.CompilerParams(use_tc_tiling_on_sc=False),
    out_type=jax.ShapeDtypeStruct((N_NODES, EMB_DIM), jnp.float32),
    scratch_types=[
        pltpu.VMEM((ROWS_PER_W,), jnp.int32),
        pltpu.VMEM((NBUF, CHUNK, EMB_DIM), jnp.float32),
        pltpu.SemaphoreType.DMA,
        pltpu.SemaphoreType.DMA,
        pltpu.SemaphoreType.DMA,
        pltpu.SemaphoreType.DMA,
    ],
)
def _gather_kernel(idx_hbm, table_hbm, out_hbm, idx_v, rows_v, g0, g1, g2, g3):
    gs = [g0, g1, g2, g3]
    wid = lax.axis_index("s") * NUM_CORES + lax.axis_index("c")
    start = wid * ROWS_PER_W
    base = jnp.minimum(start, N_NODES - ROWS_PER_W)
    pltpu.sync_copy(idx_hbm.at[pl.ds(base, ROWS_PER_W)], idx_v)
    loff = start - base

    def off(c):
        return jnp.minimum(loff + c * CHUNK, LAST_OFF)

    def fire(c, b):
        pltpu.async_copy(
            table_hbm.at[idx_v.at[pl.ds(off(c), CHUNK)]], rows_v.at[b], gs[b]
        )

    def drain(b):
        pltpu.make_async_copy(
            table_hbm.at[pl.ds(0, CHUNK), :], rows_v.at[b], gs[b]
        ).wait()

    def writeout(c, b):
        pltpu.sync_copy(rows_v.at[b], out_hbm.at[pl.ds(base + off(c), CHUNK), :])

    for b in range(NBUF):
        fire(b, b)

    @pl.loop(0, NCH - 1, step=NBUF)
    def _(j):
        for b in range(NBUF):
            c = j + b
            drain(b)
            writeout(c, b)

            @pl.when(c + NBUF < NCH)
            def _():
                fire(c + NBUF, b)

    drain(0)
    writeout(NCH - 1, 0)


def kernel(x, table):
    return _gather_kernel(x[:, 0], table)
